# baseline (speedup 1.0000x reference)
import jax
import jax.numpy as jnp
from jax import lax
from jax.experimental import pallas as pl
from jax.experimental.pallas import tpu as pltpu

_NUM_BET_BINS = 32
_D_MODEL = 128
_NUM_STREETS = 4
_ACTION_OFFSET = 10

_N = 204800
_BLK = 2048
_NQ = 4
_NI = 2 * _NQ
_G = _N // _BLK


def _start_inputs(step, masks_hbm, tok_hbm, act_hbm, st_hbm,
                  m_ring, tok_ring, act_ring, st_ring, sem_in):
    s = step % _NI
    q = step % _NQ
    src = pl.ds(step * _BLK, _BLK)
    dst = pl.ds(s * _BLK, _BLK)
    pltpu.make_async_copy(masks_hbm.at[src], m_ring.at[dst],
                          sem_in.at[q]).start()
    pltpu.make_async_copy(tok_hbm.at[src], tok_ring.at[dst],
                          sem_in.at[q]).start()
    pltpu.make_async_copy(act_hbm.at[src], act_ring.at[dst],
                          sem_in.at[q]).start()
    pltpu.make_async_copy(st_hbm.at[src], st_ring.at[dst],
                          sem_in.at[q]).start()


def _wait_inputs(step, masks_hbm, tok_hbm, act_hbm, st_hbm,
                 m_ring, tok_ring, act_ring, st_ring, sem_in):
    s = step % _NI
    q = step % _NQ
    src = pl.ds(step * _BLK, _BLK)
    dst = pl.ds(s * _BLK, _BLK)
    pltpu.make_async_copy(masks_hbm.at[src], m_ring.at[dst],
                          sem_in.at[q]).wait()
    pltpu.make_async_copy(tok_hbm.at[src], tok_ring.at[dst],
                          sem_in.at[q]).wait()
    pltpu.make_async_copy(act_hbm.at[src], act_ring.at[dst],
                          sem_in.at[q]).wait()
    pltpu.make_async_copy(st_hbm.at[src], st_ring.at[dst],
                          sem_in.at[q]).wait()


def _fused_kernel(masks_hbm, tok_hbm, act_hbm, st_hbm, table_ref, mlp_w_ref,
                  mlp_b_ref, gamma_ref, beta_ref, out_ref,
                  m_ring, tok_ring, act_ring, st_ring, scratch,
                  sem_in, sem_out):
    i = pl.program_id(0)
    q = i % _NQ
    s = i % _NI
    in_args = (masks_hbm, tok_hbm, act_hbm, st_hbm,
               m_ring, tok_ring, act_ring, st_ring, sem_in)

    @pl.when(i == 0)
    def _prologue():
        for j in range(_NQ):
            _start_inputs(j, *in_args)

    _wait_inputs(i, *in_args)

    islice = pl.ds(pl.multiple_of(s * _BLK, _BLK), _BLK)
    tok = tok_ring[islice, :]
    act = act_ring[islice, :]
    st = st_ring[islice, :]

    r = _BLK
    valid = ((tok >= _ACTION_OFFSET)
             & (tok < _ACTION_OFFSET + _NUM_BET_BINS)).astype(jnp.float32)
    aid = jnp.clip(tok - _ACTION_OFFSET, 0, _NUM_BET_BINS - 1)
    act = jnp.clip(act, 0, 1)
    st = jnp.clip(st, 0, _NUM_STREETS - 1)

    i38 = lax.broadcasted_iota(jnp.int32, (r, 38), 1)
    oh = jnp.where(i38 < 2, (act == i38).astype(jnp.float32), 0.0)
    oh = jnp.where((i38 >= 2) & (i38 < 6),
                   (st == i38 - 2).astype(jnp.float32), oh)
    oh = jnp.where(i38 >= 6, (aid == i38 - 6).astype(jnp.float32), oh)

    emb = jnp.dot(oh, table_ref[...], preferred_element_type=jnp.float32)

    h = jnp.dot(m_ring[islice, :], mlp_w_ref[...],
                preferred_element_type=jnp.float32) + mlp_b_ref[...]
    m = jnp.mean(h, axis=1, keepdims=True)
    c = h - m
    v = jnp.mean(c * c, axis=1, keepdims=True)
    h = c * lax.rsqrt(v + 1e-5) * gamma_ref[...] + beta_ref[...]
    h = jnp.maximum(h, 0.0)

    @pl.when(i >= _NQ)
    def _reclaim():
        pltpu.make_async_copy(
            scratch.at[pl.ds(q * _BLK, _BLK)],
            out_ref.at[pl.ds((i - _NQ) * _BLK, _BLK)], sem_out.at[q]).wait()

    oslice = pl.ds(pl.multiple_of(q * _BLK, _BLK), _BLK)
    scratch[oslice, :] = valid * (emb + h)

    pltpu.make_async_copy(
        scratch.at[oslice],
        out_ref.at[pl.ds(i * _BLK, _BLK)], sem_out.at[q]).start()

    @pl.when(i + _NQ < _G)
    def _prefetch():
        _start_inputs(i + _NQ, *in_args)

    @pl.when(i == _G - 1)
    def _drain():
        for j in range(_NQ):
            step = _G - 1 - j
            qq = step % _NQ
            pltpu.make_async_copy(
                scratch.at[pl.ds(qq * _BLK, _BLK)],
                out_ref.at[pl.ds(step * _BLK, _BLK)], sem_out.at[qq]).wait()


@jax.jit
def _run(token_ids, action_actors, action_streets, action_legal_masks,
         table, mlp_w, mlp_b, ln_gamma, ln_beta):
    b, l = token_ids.shape
    n = b * l
    tok = token_ids.reshape(n, 1).astype(jnp.int32)
    act = action_actors.reshape(n, 1).astype(jnp.int32)
    st = action_streets.reshape(n, 1).astype(jnp.int32)
    masks = action_legal_masks.reshape(n, _NUM_BET_BINS)

    hbm = pl.BlockSpec(memory_space=pltpu.MemorySpace.HBM)
    full = lambda shape: pl.BlockSpec(shape, lambda i: (0, 0))

    out = pl.pallas_call(
        _fused_kernel,
        grid=(_G,),
        in_specs=[
            hbm, hbm, hbm, hbm,
            full(table.shape),
            full(mlp_w.shape),
            full((1, _D_MODEL)),
            full((1, _D_MODEL)),
            full((1, _D_MODEL)),
        ],
        out_specs=pl.BlockSpec(memory_space=pltpu.MemorySpace.HBM),
        out_shape=jax.ShapeDtypeStruct((n, _D_MODEL), jnp.float32),
        scratch_shapes=[
            pltpu.VMEM((_NI * _BLK, _NUM_BET_BINS), jnp.float32),
            pltpu.VMEM((_NI * _BLK, 1), jnp.int32),
            pltpu.VMEM((_NI * _BLK, 1), jnp.int32),
            pltpu.VMEM((_NI * _BLK, 1), jnp.int32),
            pltpu.VMEM((_NQ * _BLK, _D_MODEL), jnp.float32),
            pltpu.SemaphoreType.DMA((_NQ,)),
            pltpu.SemaphoreType.DMA((_NQ,)),
        ],
    )(masks, tok, act, st, table, mlp_w,
      mlp_b.reshape(1, _D_MODEL), ln_gamma.reshape(1, _D_MODEL),
      ln_beta.reshape(1, _D_MODEL))
    return out.reshape(b, l, _D_MODEL)


def kernel(token_ids, action_actors, action_streets, action_legal_masks,
           actor_emb_w, street_emb_w, action_type_emb_w, mlp_w, mlp_b,
           ln_gamma, ln_beta):
    table = jnp.concatenate([actor_emb_w, street_emb_w, action_type_emb_w],
                            axis=0)
    return _run(token_ids, action_actors, action_streets, action_legal_masks,
                table, mlp_w, mlp_b, ln_gamma, ln_beta)


# speedup vs baseline: 1.0374x; 1.0374x over previous
import jax
import jax.numpy as jnp
from jax import lax
from jax.experimental import pallas as pl
from jax.experimental.pallas import tpu as pltpu

_NUM_BET_BINS = 32
_D_MODEL = 128
_NUM_STREETS = 4
_ACTION_OFFSET = 10

_N = 204800
_BLK = 2048
_NQ = 4
_NI = 2 * _NQ
_G = _N // _BLK


def _start_inputs(step, masks_hbm, tok_hbm, act_hbm, st_hbm,
                  m_ring, tok_ring, act_ring, st_ring, sem_in):
    s = step % _NI
    q = step % _NQ
    src = pl.ds(step * _BLK, _BLK)
    dst = pl.ds(s * _BLK, _BLK)
    pltpu.make_async_copy(masks_hbm.at[src], m_ring.at[dst],
                          sem_in.at[q]).start()
    pltpu.make_async_copy(tok_hbm.at[src], tok_ring.at[dst],
                          sem_in.at[q]).start()
    pltpu.make_async_copy(act_hbm.at[src], act_ring.at[dst],
                          sem_in.at[q]).start()
    pltpu.make_async_copy(st_hbm.at[src], st_ring.at[dst],
                          sem_in.at[q]).start()


def _wait_inputs(step, masks_hbm, tok_hbm, act_hbm, st_hbm,
                 m_ring, tok_ring, act_ring, st_ring, sem_in):
    s = step % _NI
    q = step % _NQ
    src = pl.ds(step * _BLK, _BLK)
    dst = pl.ds(s * _BLK, _BLK)
    pltpu.make_async_copy(masks_hbm.at[src], m_ring.at[dst],
                          sem_in.at[q]).wait()
    pltpu.make_async_copy(tok_hbm.at[src], tok_ring.at[dst],
                          sem_in.at[q]).wait()
    pltpu.make_async_copy(act_hbm.at[src], act_ring.at[dst],
                          sem_in.at[q]).wait()
    pltpu.make_async_copy(st_hbm.at[src], st_ring.at[dst],
                          sem_in.at[q]).wait()


def _fused_kernel(masks_hbm, tok_hbm, act_hbm, st_hbm, table_ref, mlp_w_ref,
                  mlp_b_ref, gamma_ref, beta_ref, out_ref,
                  m_ring, tok_ring, act_ring, st_ring, scratch,
                  sem_in, sem_out):
    i = pl.program_id(0)
    q = i % _NQ
    s = i % _NI
    in_args = (masks_hbm, tok_hbm, act_hbm, st_hbm,
               m_ring, tok_ring, act_ring, st_ring, sem_in)

    @pl.when(i == 0)
    def _prologue():
        for j in range(_NQ):
            _start_inputs(j, *in_args)

    _wait_inputs(i, *in_args)

    islice = pl.ds(pl.multiple_of(s * _BLK, _BLK), _BLK)
    tok = tok_ring[islice, :]
    act = act_ring[islice, :]
    st = st_ring[islice, :]

    r = _BLK
    valid = ((tok >= _ACTION_OFFSET)
             & (tok < _ACTION_OFFSET + _NUM_BET_BINS)).astype(jnp.float32)
    aid = jnp.clip(tok - _ACTION_OFFSET, 0, _NUM_BET_BINS - 1)
    act = jnp.clip(act, 0, 1)
    st = jnp.clip(st, 0, _NUM_STREETS - 1)

    oh_a = (act == lax.broadcasted_iota(jnp.int32, (r, 2), 1)
            ).astype(jnp.float32)
    oh_s = (st == lax.broadcasted_iota(jnp.int32, (r, _NUM_STREETS), 1)
            ).astype(jnp.float32)
    oh_t = (aid == lax.broadcasted_iota(jnp.int32, (r, _NUM_BET_BINS), 1)
            ).astype(jnp.float32)
    emb = (jnp.dot(oh_a, table_ref[0:2, :],
                   preferred_element_type=jnp.float32)
           + jnp.dot(oh_s, table_ref[2:6, :],
                     preferred_element_type=jnp.float32)
           + jnp.dot(oh_t, table_ref[6:38, :],
                     preferred_element_type=jnp.float32))

    h = jnp.dot(m_ring[islice, :], mlp_w_ref[...],
                preferred_element_type=jnp.float32) + mlp_b_ref[...]
    m = jnp.mean(h, axis=1, keepdims=True)
    c = h - m
    v = jnp.mean(c * c, axis=1, keepdims=True)
    h = c * lax.rsqrt(v + 1e-5) * gamma_ref[...] + beta_ref[...]
    h = jnp.maximum(h, 0.0)

    @pl.when(i >= _NQ)
    def _reclaim():
        pltpu.make_async_copy(
            scratch.at[pl.ds(q * _BLK, _BLK)],
            out_ref.at[pl.ds((i - _NQ) * _BLK, _BLK)], sem_out.at[q]).wait()

    oslice = pl.ds(pl.multiple_of(q * _BLK, _BLK), _BLK)
    scratch[oslice, :] = valid * (emb + h)

    pltpu.make_async_copy(
        scratch.at[oslice],
        out_ref.at[pl.ds(i * _BLK, _BLK)], sem_out.at[q]).start()

    @pl.when(i + _NQ < _G)
    def _prefetch():
        _start_inputs(i + _NQ, *in_args)

    @pl.when(i == _G - 1)
    def _drain():
        for j in range(_NQ):
            step = _G - 1 - j
            qq = step % _NQ
            pltpu.make_async_copy(
                scratch.at[pl.ds(qq * _BLK, _BLK)],
                out_ref.at[pl.ds(step * _BLK, _BLK)], sem_out.at[qq]).wait()


@jax.jit
def _run(token_ids, action_actors, action_streets, action_legal_masks,
         table, mlp_w, mlp_b, ln_gamma, ln_beta):
    b, l = token_ids.shape
    n = b * l
    tok = token_ids.reshape(n, 1).astype(jnp.int32)
    act = action_actors.reshape(n, 1).astype(jnp.int32)
    st = action_streets.reshape(n, 1).astype(jnp.int32)
    masks = action_legal_masks.reshape(n, _NUM_BET_BINS)

    hbm = pl.BlockSpec(memory_space=pltpu.MemorySpace.HBM)
    full = lambda shape: pl.BlockSpec(shape, lambda i: (0, 0))

    out = pl.pallas_call(
        _fused_kernel,
        grid=(_G,),
        in_specs=[
            hbm, hbm, hbm, hbm,
            full(table.shape),
            full(mlp_w.shape),
            full((1, _D_MODEL)),
            full((1, _D_MODEL)),
            full((1, _D_MODEL)),
        ],
        out_specs=pl.BlockSpec(memory_space=pltpu.MemorySpace.HBM),
        out_shape=jax.ShapeDtypeStruct((n, _D_MODEL), jnp.float32),
        scratch_shapes=[
            pltpu.VMEM((_NI * _BLK, _NUM_BET_BINS), jnp.float32),
            pltpu.VMEM((_NI * _BLK, 1), jnp.int32),
            pltpu.VMEM((_NI * _BLK, 1), jnp.int32),
            pltpu.VMEM((_NI * _BLK, 1), jnp.int32),
            pltpu.VMEM((_NQ * _BLK, _D_MODEL), jnp.float32),
            pltpu.SemaphoreType.DMA((_NQ,)),
            pltpu.SemaphoreType.DMA((_NQ,)),
        ],
    )(masks, tok, act, st, table, mlp_w,
      mlp_b.reshape(1, _D_MODEL), ln_gamma.reshape(1, _D_MODEL),
      ln_beta.reshape(1, _D_MODEL))
    return out.reshape(b, l, _D_MODEL)


def kernel(token_ids, action_actors, action_streets, action_legal_masks,
           actor_emb_w, street_emb_w, action_type_emb_w, mlp_w, mlp_b,
           ln_gamma, ln_beta):
    table = jnp.concatenate([actor_emb_w, street_emb_w, action_type_emb_w],
                            axis=0)
    return _run(token_ids, action_actors, action_streets, action_legal_masks,
                table, mlp_w, mlp_b, ln_gamma, ln_beta)
